# baseline (speedup 1.0000x reference)
import functools

import jax
import jax.numpy as jnp
from jax import lax
from jax.experimental import pallas as pl
from jax.experimental.pallas import tpu as pltpu
from jax.experimental.pallas import tpu_sc as plsc

_BM = 1024
_CH = 128


def _row_index_kernel(n, a_ref, out_ref):
    i = pl.program_id(0)
    w_all = a_ref.bitcast(jnp.int32)[...]
    pm = w_all.shape[0]
    lane = lax.broadcasted_iota(jnp.int32, (pm, 128), 1)
    acc_l = jnp.zeros((pm, 128), jnp.int32)
    acc_v = jnp.zeros((pm, 128), jnp.int32)
    for v in range(n // 128):
        w = lax.slice_in_dim(w_all, v * 128, (v + 1) * 128, axis=1)
        acc_l = acc_l + w * lane
        acc_v = acc_v + w * v
    sl = jnp.sum(acc_l, axis=1)
    sv = jnp.sum(acc_v, axis=1)
    base = i * _BM // n * n
    for k in range(4):
        lo = (sl >> (8 * k)) & 255
        hi = (sv >> (8 * k)) & 255
        out_ref[0, k, :] = (hi << 7) + lo + base


def _extract_indices(a_b, n):
    rows, _ = a_b.shape
    nb = rows // _BM
    out = pl.pallas_call(
        functools.partial(_row_index_kernel, n),
        grid=(nb,),
        in_specs=[pl.BlockSpec((_BM, n), lambda i: (i, 0))],
        out_specs=pl.BlockSpec((1, 4, _BM // 4), lambda i: (i, 0, 0)),
        out_shape=jax.ShapeDtypeStruct((nb, 4, _BM // 4), jnp.int32),
        compiler_params=pltpu.CompilerParams(
            dimension_semantics=("parallel",)),
    )(a_b.view(jnp.int8))
    return out.transpose(0, 2, 1).reshape(rows)


def _sc_gather(table, idx2d):
    rows, d = table.shape
    info = plsc.get_sparse_core_info()
    nc, ns = info.num_cores, info.num_subcores
    nw = nc * ns
    per_w = rows // nw
    k = per_w // _CH
    mesh = plsc.VectorSubcoreMesh(core_axis_name="c", subcore_axis_name="s")

    @functools.partial(
        pl.kernel,
        mesh=mesh,
        out_type=jax.ShapeDtypeStruct((rows, d), jnp.float32),
        scratch_types=[
            pltpu.VMEM((k, _CH), jnp.int32),
            pltpu.VMEM((per_w, d), jnp.float32),
            pltpu.SemaphoreType.DMA,
        ],
    )
    def run(tab_hbm, idx_hbm, out_hbm, idx_v, rows_v, sem):
        wid = lax.axis_index("s") * nc + lax.axis_index("c")
        base = wid * per_w
        pltpu.sync_copy(idx_hbm.at[pl.ds(wid * k, k)], idx_v)
        copies = [
            pltpu.async_copy(tab_hbm.at[idx_v.at[j]],
                             rows_v.at[pl.ds(j * _CH, _CH)], sem)
            for j in range(k)
        ]
        for c in copies:
            c.wait()
        pltpu.sync_copy(rows_v, out_hbm.at[pl.ds(base, per_w)])

    return run(table, idx2d)


def kernel(entities, assignments):
    b, n, d = entities.shape
    flat_idx = _extract_indices(assignments.reshape(b * n, n), n)
    out = _sc_gather(entities.reshape(b * n, d), flat_idx.reshape(-1, _CH))
    return out.reshape(b, n, d)


# speedup vs baseline: 1.0530x; 1.0530x over previous
import functools

import jax
import jax.numpy as jnp
from jax import lax
from jax.experimental import pallas as pl
from jax.experimental.pallas import tpu as pltpu
from jax.experimental.pallas import tpu_sc as plsc

_BM = 2048
_CH = 128


def _row_index_kernel(n, a_ref, out_ref):
    i = pl.program_id(0)
    w_all = a_ref.bitcast(jnp.int32)[...]
    pm = w_all.shape[0]
    lane = lax.broadcasted_iota(jnp.int32, (pm, 128), 1)
    acc_l = jnp.zeros((pm, 128), jnp.int32)
    acc_v = jnp.zeros((pm, 128), jnp.int32)
    for v in range(n // 128):
        w = lax.slice_in_dim(w_all, v * 128, (v + 1) * 128, axis=1)
        acc_l = acc_l + w * lane
        acc_v = acc_v + w * v
    sl = jnp.sum(acc_l, axis=1)
    sv = jnp.sum(acc_v, axis=1)
    base = i * _BM // n * n
    for k in range(4):
        lo = (sl >> (8 * k)) & 255
        hi = (sv >> (8 * k)) & 255
        out_ref[0, k, :] = (hi << 7) + lo + base


def _extract_indices(a_b, n):
    rows, _ = a_b.shape
    nb = rows // _BM
    out = pl.pallas_call(
        functools.partial(_row_index_kernel, n),
        grid=(nb,),
        in_specs=[pl.BlockSpec((_BM, n), lambda i: (i, 0))],
        out_specs=pl.BlockSpec((1, 4, _BM // 4), lambda i: (i, 0, 0)),
        out_shape=jax.ShapeDtypeStruct((nb, 4, _BM // 4), jnp.int32),
        compiler_params=pltpu.CompilerParams(
            dimension_semantics=("parallel",)),
    )(a_b.view(jnp.int8))
    return out.transpose(0, 2, 1).reshape(rows)


def _sc_gather(table, idx2d):
    rows, d = table.shape
    info = plsc.get_sparse_core_info()
    nc, ns = info.num_cores, info.num_subcores
    nw = nc * ns
    per_w = rows // nw
    k = per_w // _CH
    mesh = plsc.VectorSubcoreMesh(core_axis_name="c", subcore_axis_name="s")

    @functools.partial(
        pl.kernel,
        mesh=mesh,
        out_type=jax.ShapeDtypeStruct((rows, d), jnp.float32),
        scratch_types=[
            pltpu.VMEM((k, _CH), jnp.int32),
            pltpu.VMEM((per_w, d), jnp.float32),
            pltpu.SemaphoreType.DMA,
        ],
    )
    def run(tab_hbm, idx_hbm, out_hbm, idx_v, rows_v, sem):
        wid = lax.axis_index("s") * nc + lax.axis_index("c")
        base = wid * per_w
        pltpu.sync_copy(idx_hbm.at[pl.ds(wid * k, k)], idx_v)
        copies = [
            pltpu.async_copy(tab_hbm.at[idx_v.at[j]],
                             rows_v.at[pl.ds(j * _CH, _CH)], sem)
            for j in range(k)
        ]
        for c in copies:
            c.wait()
        pltpu.sync_copy(rows_v, out_hbm.at[pl.ds(base, per_w)])

    return run(table, idx2d)


def kernel(entities, assignments):
    b, n, d = entities.shape
    flat_idx = _extract_indices(assignments.reshape(b * n, n), n)
    out = _sc_gather(entities.reshape(b * n, d), flat_idx.reshape(-1, _CH))
    return out.reshape(b, n, d)


# speedup vs baseline: 1.0558x; 1.0026x over previous
import functools

import jax
import jax.numpy as jnp
from jax import lax
from jax.experimental import pallas as pl
from jax.experimental.pallas import tpu as pltpu
from jax.experimental.pallas import tpu_sc as plsc

_BM = 4096
_CH = 128


def _row_index_kernel(n, a_ref, out_ref):
    i = pl.program_id(0)
    w_all = a_ref.bitcast(jnp.int32)[...]
    pm = w_all.shape[0]
    lane = lax.broadcasted_iota(jnp.int32, (pm, 128), 1)
    acc_l = jnp.zeros((pm, 128), jnp.int32)
    acc_v = jnp.zeros((pm, 128), jnp.int32)
    for v in range(n // 128):
        w = lax.slice_in_dim(w_all, v * 128, (v + 1) * 128, axis=1)
        acc_l = acc_l + w * lane
        acc_v = acc_v + w * v
    sl = jnp.sum(acc_l, axis=1)
    sv = jnp.sum(acc_v, axis=1)
    base = (i * _BM + 4 * lax.iota(jnp.int32, pm)) // n * n
    for k in range(4):
        lo = (sl >> (8 * k)) & 255
        hi = (sv >> (8 * k)) & 255
        out_ref[0, k, :] = (hi << 7) + lo + base


def _extract_indices(a_b, n):
    rows, _ = a_b.shape
    nb = rows // _BM
    out = pl.pallas_call(
        functools.partial(_row_index_kernel, n),
        grid=(nb,),
        in_specs=[pl.BlockSpec((_BM, n), lambda i: (i, 0))],
        out_specs=pl.BlockSpec((1, 4, _BM // 4), lambda i: (i, 0, 0)),
        out_shape=jax.ShapeDtypeStruct((nb, 4, _BM // 4), jnp.int32),
        compiler_params=pltpu.CompilerParams(
            dimension_semantics=("parallel",)),
    )(a_b.view(jnp.int8))
    return out.transpose(0, 2, 1).reshape(rows)


def _sc_gather(table, idx2d):
    rows, d = table.shape
    info = plsc.get_sparse_core_info()
    nc, ns = info.num_cores, info.num_subcores
    nw = nc * ns
    per_w = rows // nw
    k = per_w // _CH
    mesh = plsc.VectorSubcoreMesh(core_axis_name="c", subcore_axis_name="s")

    @functools.partial(
        pl.kernel,
        mesh=mesh,
        out_type=jax.ShapeDtypeStruct((rows, d), jnp.float32),
        scratch_types=[
            pltpu.VMEM((k, _CH), jnp.int32),
            pltpu.VMEM((per_w, d), jnp.float32),
            pltpu.SemaphoreType.DMA,
        ],
    )
    def run(tab_hbm, idx_hbm, out_hbm, idx_v, rows_v, sem):
        wid = lax.axis_index("s") * nc + lax.axis_index("c")
        base = wid * per_w
        pltpu.sync_copy(idx_hbm.at[pl.ds(wid * k, k)], idx_v)
        copies = [
            pltpu.async_copy(tab_hbm.at[idx_v.at[j]],
                             rows_v.at[pl.ds(j * _CH, _CH)], sem)
            for j in range(k)
        ]
        for c in copies:
            c.wait()
        pltpu.sync_copy(rows_v, out_hbm.at[pl.ds(base, per_w)])

    return run(table, idx2d)


def kernel(entities, assignments):
    b, n, d = entities.shape
    flat_idx = _extract_indices(assignments.reshape(b * n, n), n)
    out = _sc_gather(entities.reshape(b * n, d), flat_idx.reshape(-1, _CH))
    return out.reshape(b, n, d)


# speedup vs baseline: 1.0608x; 1.0048x over previous
import functools

import jax
import jax.numpy as jnp
from jax import lax
from jax.experimental import pallas as pl
from jax.experimental.pallas import tpu as pltpu
from jax.experimental.pallas import tpu_sc as plsc

_BM = 4096
_CH = 128


def _row_index_kernel(n, a0_ref, a1_ref, out_ref):
    i = pl.program_id(0)
    w0 = a0_ref.bitcast(jnp.int32)[...]
    w1 = a1_ref.bitcast(jnp.int32)[...]
    pm = w0.shape[0]
    lane = lax.broadcasted_iota(jnp.int32, (pm, 128), 1)
    acc_l = jnp.zeros((pm, 128), jnp.int32)
    acc_v = jnp.zeros((pm, 128), jnp.int32)
    half = n // 256
    for v in range(n // 128):
        w_all = w0 if v < half else w1
        w = lax.slice_in_dim(w_all, (v % half) * 128, (v % half + 1) * 128,
                             axis=1)
        acc_l = acc_l + w * lane
        acc_v = acc_v + w * v
    sl = jnp.sum(acc_l, axis=1)
    sv = jnp.sum(acc_v, axis=1)
    base = (i * _BM + 4 * lax.iota(jnp.int32, pm)) // n * n
    for k in range(4):
        lo = (sl >> (8 * k)) & 255
        hi = (sv >> (8 * k)) & 255
        out_ref[0, k, :] = (hi << 7) + lo + base


def _extract_indices(a_b, n):
    rows, _ = a_b.shape
    nb = rows // _BM
    out = pl.pallas_call(
        functools.partial(_row_index_kernel, n),
        grid=(nb,),
        in_specs=[pl.BlockSpec((_BM, n // 2), lambda i: (i, 0)),
                  pl.BlockSpec((_BM, n // 2), lambda i: (i, 1))],
        out_specs=pl.BlockSpec((1, 4, _BM // 4), lambda i: (i, 0, 0)),
        out_shape=jax.ShapeDtypeStruct((nb, 4, _BM // 4), jnp.int32),
        compiler_params=pltpu.CompilerParams(
            dimension_semantics=("parallel",)),
    )(a_b.view(jnp.int8), a_b.view(jnp.int8))
    return out.transpose(0, 2, 1).reshape(rows)


def _sc_gather(table, idx2d):
    rows, d = table.shape
    info = plsc.get_sparse_core_info()
    nc, ns = info.num_cores, info.num_subcores
    nw = nc * ns
    per_w = rows // nw
    k = per_w // _CH
    mesh = plsc.VectorSubcoreMesh(core_axis_name="c", subcore_axis_name="s")

    @functools.partial(
        pl.kernel,
        mesh=mesh,
        out_type=jax.ShapeDtypeStruct((rows, d), jnp.float32),
        scratch_types=[
            pltpu.VMEM((k, _CH), jnp.int32),
            pltpu.VMEM((per_w, d), jnp.float32),
            pltpu.SemaphoreType.DMA,
        ],
    )
    def run(tab_hbm, idx_hbm, out_hbm, idx_v, rows_v, sem):
        wid = lax.axis_index("s") * nc + lax.axis_index("c")
        base = wid * per_w
        pltpu.sync_copy(idx_hbm.at[pl.ds(wid * k, k)], idx_v)
        copies = [
            pltpu.async_copy(tab_hbm.at[idx_v.at[j]],
                             rows_v.at[pl.ds(j * _CH, _CH)], sem)
            for j in range(k)
        ]
        for c in copies:
            c.wait()
        pltpu.sync_copy(rows_v, out_hbm.at[pl.ds(base, per_w)])

    return run(table, idx2d)


def kernel(entities, assignments):
    b, n, d = entities.shape
    flat_idx = _extract_indices(assignments.reshape(b * n, n), n)
    out = _sc_gather(entities.reshape(b * n, d), flat_idx.reshape(-1, _CH))
    return out.reshape(b, n, d)
